# baseline (speedup 1.0000x reference)
import functools

import jax
import jax.numpy as jnp
import numpy as np
from jax import lax
from jax.experimental import pallas as pl
from jax.experimental.pallas import tpu as pltpu
from jax.experimental.pallas import tpu_sc as plsc

VIEWS = 128
NDET = 512
C = 8
K7 = 7
VU = VIEWS * NDET
N = 64 * 64 * VIEWS
HID = K7 * C * 2
OUTC = K7 * C
COLS = 64


VB = 8
ROWS = VB * NDET


def _gelu_exact(x):
    return 0.5 * x * (1.0 + lax.erf(x * np.float32(1.0 / np.sqrt(2.0))))


def _conv_body(xt_ref, w1_ref, b1_ref, w2_ref, b2_ref, out_ref):
    x2 = xt_ref[...].reshape(ROWS, C)
    i = lax.broadcasted_iota(jnp.int32, (ROWS, 1), 0)
    first = (i % NDET) == 0
    last = (i % NDET) == (NDET - 1)

    def shifts(v):
        z = jnp.zeros((1, v.shape[1]), jnp.float32)
        vm = jnp.where(first, 0.0, jnp.concatenate([z, v[:-1]], axis=0))
        vp = jnp.where(last, 0.0, jnp.concatenate([v[1:], z], axis=0))
        return vm, vp

    xm, xp = shifts(x2)
    f32 = jnp.float32
    xcat = jnp.concatenate([xm, x2, xp], axis=1)
    h = jnp.dot(xcat, w1_ref[...].reshape(3 * C, HID),
                preferred_element_type=f32) + b1_ref[...]
    h = _gelu_exact(h)
    hm, hp = shifts(h)
    hcat = jnp.concatenate([hm, h, hp], axis=1)
    y = jnp.dot(hcat, w2_ref[...].reshape(3 * HID, COLS),
                preferred_element_type=f32) + b2_ref[...]
    out_ref[...] = y


def _pack_body(lo_ref, hi_ref, out_ref):
    lob = lax.bitcast_convert_type(lo_ref[...], jnp.uint32)
    hib = lax.bitcast_convert_type(hi_ref[...], jnp.uint32)

    def rb(b):
        return (b + jnp.uint32(0x7FFF) + ((b >> 16) & jnp.uint32(1))) >> 16

    word = (rb(hib) << 16) | rb(lob)
    out_ref[...] = lax.bitcast_convert_type(word, jnp.int32)


def _make_pair_table(table, *, interpret=False):
    tnext = jnp.concatenate([table[1:], table[-1:]], axis=0)
    return pl.pallas_call(
        _pack_body,
        grid=(VIEWS // VB,),
        in_specs=[
            pl.BlockSpec((ROWS, COLS), lambda i: (i, 0)),
            pl.BlockSpec((ROWS, COLS), lambda i: (i, 0)),
        ],
        out_specs=pl.BlockSpec((ROWS, COLS), lambda i: (i, 0)),
        out_shape=jax.ShapeDtypeStruct((VU, COLS), jnp.int32),
        interpret=interpret,
    )(table, tnext)


def _make_table(xt, w1t, b1, w2p, b2p, *, interpret=False):
    return pl.pallas_call(
        _conv_body,
        grid=(VIEWS // VB,),
        in_specs=[
            pl.BlockSpec((VB, NDET, C), lambda i: (i, 0, 0)),
            pl.BlockSpec((3, C, HID), lambda i: (0, 0, 0)),
            pl.BlockSpec((1, HID), lambda i: (0, 0)),
            pl.BlockSpec((3, HID, COLS), lambda i: (0, 0, 0)),
            pl.BlockSpec((1, COLS), lambda i: (0, 0)),
        ],
        out_specs=pl.BlockSpec((ROWS, COLS), lambda i: (i, 0)),
        out_shape=jax.ShapeDtypeStruct((VU, COLS), jnp.float32),
        interpret=interpret,
    )(xt, w1t, b1, w2p, b2p)


WR = 8
WCOL = 4096


def _wq_body(idx_ref, low_ref, wq_ref):
    idx = idx_ref[...]
    f = jnp.floor(idx)
    w = idx - f
    low_ref[...] = f.astype(jnp.int32)
    u = w - 1.0
    cw, sw = jnp.cos(w), jnp.sin(w)
    cu, su = jnp.cos(u), jnp.sin(u)

    def harmonics(cc, ss):
        c2 = 2.0 * cc * cc - 1.0
        s2 = 2.0 * ss * cc
        c3 = c2 * cc - s2 * ss
        s3 = s2 * cc + c2 * ss
        return c2, s2, c3, s3

    c2w, s2w, c3w, s3w = harmonics(cw, sw)
    c2u, s2u, c3u, s3u = harmonics(cu, su)
    wl = 1.0 - w
    wh = w
    z = jnp.zeros_like(w)
    rows = [wl, wl * cw, wl * sw, wl * c2w, wl * s2w, wl * c3w, wl * s3w, z,
            wh, wh * cu, wh * su, wh * c2u, wh * s2u, wh * c3u, wh * s3u, z]
    for k in range(16):
        wq_ref[:, k, :] = rows[k].reshape(WR * WCOL // CHW, CHW)


CHW = 512


def _make_wq(idx2, *, interpret=False):
    nrow = N // WCOL
    nch = WR * WCOL // CHW
    return pl.pallas_call(
        _wq_body,
        grid=(nrow // WR,),
        in_specs=[pl.BlockSpec((WR, WCOL), lambda i: (i, 0))],
        out_specs=[
            pl.BlockSpec((WR, WCOL), lambda i: (i, 0)),
            pl.BlockSpec((nch, 16, CHW), lambda i: (i, 0, 0)),
        ],
        out_shape=[
            jax.ShapeDtypeStruct((nrow, WCOL), jnp.int32),
            jax.ShapeDtypeStruct((N // CHW, 16, CHW), jnp.float32),
        ],
        interpret=interpret,
    )(idx2)


NW = 32
NT = N // NW
CH = 512
NCHUNK = NT // CH
NG = CH // 16
NB = CH // 128


def _sc_body(table, lowidx, wq, out_hbm,
             idxlo_v, rows_v0, rows_v1, wq_v0, wq_v1, out_v0, out_v1,
             gsem0, gsem1, osem0, osem1):
    wid = lax.axis_index("s") * 2 + lax.axis_index("c")
    tbase = wid * NT
    trows = NT // 128
    pltpu.sync_copy(lowidx.at[pl.ds(wid * trows, trows)], idxlo_v)

    slots = ((rows_v0, wq_v0, out_v0, gsem0, osem0),
             (rows_v1, wq_v1, out_v1, gsem1, osem1))

    def in_copies(ci, slot):
        rows_v, wq_v, _, gsem, _ = slots[slot]
        gci = wid * NCHUNK + ci
        cps = [pltpu.make_async_copy(wq.at[gci], wq_v, gsem)]
        for j in range(NB):
            cps.append(pltpu.make_async_copy(
                table.at[idxlo_v.at[ci * NB + j]],
                rows_v.at[pl.ds(j * 128, 128)], gsem))
        return cps

    def fire_in(ci, slot):
        for cp in in_copies(ci, slot):
            cp.start()

    def wait_in(ci, slot):
        for cp in in_copies(ci, slot):
            cp.wait()

    def out_copy(ci, slot):
        _, _, out_v, _, osem = slots[slot]
        gci = wid * NCHUNK + ci
        return pltpu.make_async_copy(out_v, out_hbm.at[gci], osem)

    def chunk_compute(ci, slot):
        rows_v, wq_v, out_v, _, _ = slots[slot]

        @plsc.parallel_loop(0, NG)
        def group(g):
            g16 = pl.multiple_of(g * 16, 16)
            iota = lax.iota(jnp.int32, 16)
            riv = iota + g16
            wls = [wq_v[m, pl.ds(g16, 16)] for m in range(K7)]
            whs = [wq_v[8 + m, pl.ds(g16, 16)] for m in range(K7)]
            for d in range(C):
                av = (iota + d) & 7
                av8 = av * 8
                acc = None
                for m in range(K7):
                    pw = plsc.load_gather(rows_v, [riv, av8 + m])
                    bf = plsc.bitcast(pw, jnp.bfloat16)
                    vlo, vhi = plsc.unpack(bf,
                                           format=plsc.PackFormat.INTERLEAVED)
                    t = vlo * wls[m] + vhi * whs[m]
                    acc = t if acc is None else acc + t
                plsc.store_scatter(out_v, [av, riv], acc)

    fire_in(0, 0)

    def pair(ci2, carry):
        for b in (0, 1):
            ci = ci2 * 2 + b
            wait_in(ci, b)
            fire_in(jnp.minimum(ci + 1, NCHUNK - 1), 1 - b)

            @pl.when(ci2 >= 1)
            def _drain_out():
                out_copy(ci, b).wait()

            chunk_compute(ci, b)
            out_copy(ci, b).start()
        return carry

    lax.fori_loop(0, NCHUNK // 2, pair, 0)
    wait_in(NCHUNK - 1, 0)
    out_copy(NCHUNK - 2, 0).wait()
    out_copy(NCHUNK - 1, 1).wait()


def _sc_interp(table, lowidx2, wq4):
    mesh = plsc.VectorSubcoreMesh(core_axis_name="c", subcore_axis_name="s")
    f = functools.partial(
        pl.kernel, mesh=mesh,
        compiler_params=pltpu.CompilerParams(needs_layout_passes=False,
                                             use_tc_tiling_on_sc=False),
        out_type=jax.ShapeDtypeStruct((N // CH, C, CH), jnp.float32),
        scratch_types=[
            pltpu.VMEM((NT // 128, 128), jnp.int32),
            pltpu.VMEM((CH, COLS), jnp.int32),
            pltpu.VMEM((CH, COLS), jnp.int32),
            pltpu.VMEM((16, CH), jnp.float32),
            pltpu.VMEM((16, CH), jnp.float32),
            pltpu.VMEM((C, CH), jnp.float32),
            pltpu.VMEM((C, CH), jnp.float32),
            pltpu.SemaphoreType.DMA,
            pltpu.SemaphoreType.DMA,
            pltpu.SemaphoreType.DMA,
            pltpu.SemaphoreType.DMA,
        ],
    )(_sc_body)
    return f(table, lowidx2, wq4)


def kernel(x, W1, b1, W2, b2, indices):
    xt = jnp.transpose(x[0], (1, 2, 0))
    w1t = jnp.transpose(W1, (2, 1, 0))
    o = np.arange(OUTC)
    cols = (o // K7) * 8 + (o % K7)
    w2t = jnp.transpose(W2, (2, 1, 0))
    w2p = jnp.zeros((3, HID, COLS), jnp.float32).at[:, :, cols].set(w2t)
    b2p = jnp.zeros((COLS,), jnp.float32).at[cols].set(b2)

    table = _make_table(xt, w1t, b1[None, :], w2p, b2p[None, :])
    tablep = _make_pair_table(table)
    low, wq4 = _make_wq(indices.reshape(N // WCOL, WCOL))
    out4 = _sc_interp(tablep, low.reshape(N // 128, 128), wq4)
    return out4.transpose(1, 0, 2).reshape(1, C, N // VIEWS, VIEWS)


# speedup vs baseline: 1.1632x; 1.1632x over previous
import functools

import jax
import jax.numpy as jnp
import numpy as np
from jax import lax
from jax.experimental import pallas as pl
from jax.experimental.pallas import tpu as pltpu
from jax.experimental.pallas import tpu_sc as plsc

VIEWS = 128
NDET = 512
C = 8
K7 = 7
VU = VIEWS * NDET
N = 64 * 64 * VIEWS
HID = K7 * C * 2
OUTC = K7 * C
COLS = 64


VB = 8
ROWS = VB * NDET


def _gelu_exact(x):
    return 0.5 * x * (1.0 + lax.erf(x * np.float32(1.0 / np.sqrt(2.0))))


def _conv_body(xt_ref, w1_ref, b1_ref, w2_ref, b2_ref, out_ref):
    x2 = xt_ref[...].reshape(ROWS, C)
    i = lax.broadcasted_iota(jnp.int32, (ROWS, 1), 0)
    first = (i % NDET) == 0
    last = (i % NDET) == (NDET - 1)

    def shifts(v):
        z = jnp.zeros((1, v.shape[1]), jnp.float32)
        vm = jnp.where(first, 0.0, jnp.concatenate([z, v[:-1]], axis=0))
        vp = jnp.where(last, 0.0, jnp.concatenate([v[1:], z], axis=0))
        return vm, vp

    xm, xp = shifts(x2)
    f32 = jnp.float32
    xcat = jnp.concatenate([xm, x2, xp], axis=1)
    h = jnp.dot(xcat, w1_ref[...].reshape(3 * C, HID),
                preferred_element_type=f32) + b1_ref[...]
    h = _gelu_exact(h)
    hm, hp = shifts(h)
    hcat = jnp.concatenate([hm, h, hp], axis=1)
    y = jnp.dot(hcat, w2_ref[...].reshape(3 * HID, COLS),
                preferred_element_type=f32) + b2_ref[...]
    out_ref[...] = y


def _pack_body(lo_ref, hi_ref, out_ref):
    lob = lax.bitcast_convert_type(lo_ref[...], jnp.uint32)
    hib = lax.bitcast_convert_type(hi_ref[...], jnp.uint32)

    def rb(b):
        return (b + jnp.uint32(0x7FFF) + ((b >> 16) & jnp.uint32(1))) >> 16

    word = (rb(hib) << 16) | rb(lob)
    out_ref[...] = lax.bitcast_convert_type(word, jnp.int32)


def _make_pair_table(table, *, interpret=False):
    tnext = jnp.concatenate([table[1:], table[-1:]], axis=0)
    return pl.pallas_call(
        _pack_body,
        grid=(VIEWS // VB,),
        in_specs=[
            pl.BlockSpec((ROWS, COLS), lambda i: (i, 0)),
            pl.BlockSpec((ROWS, COLS), lambda i: (i, 0)),
        ],
        out_specs=pl.BlockSpec((ROWS, COLS), lambda i: (i, 0)),
        out_shape=jax.ShapeDtypeStruct((VU, COLS), jnp.int32),
        interpret=interpret,
    )(table, tnext)


def _make_table(xt, w1t, b1, w2p, b2p, *, interpret=False):
    return pl.pallas_call(
        _conv_body,
        grid=(VIEWS // VB,),
        in_specs=[
            pl.BlockSpec((VB, NDET, C), lambda i: (i, 0, 0)),
            pl.BlockSpec((3, C, HID), lambda i: (0, 0, 0)),
            pl.BlockSpec((1, HID), lambda i: (0, 0)),
            pl.BlockSpec((3, HID, COLS), lambda i: (0, 0, 0)),
            pl.BlockSpec((1, COLS), lambda i: (0, 0)),
        ],
        out_specs=pl.BlockSpec((ROWS, COLS), lambda i: (i, 0)),
        out_shape=jax.ShapeDtypeStruct((VU, COLS), jnp.float32),
        interpret=interpret,
    )(xt, w1t, b1, w2p, b2p)


WR = 8
WCOL = 4096


def _wq_body(idx_ref, low_ref, wq_ref):
    idx = idx_ref[...]
    f = jnp.floor(idx)
    w = idx - f
    low_ref[...] = f.astype(jnp.int32)
    u = w - 1.0
    cw, sw = jnp.cos(w), jnp.sin(w)
    cu, su = jnp.cos(u), jnp.sin(u)

    def harmonics(cc, ss):
        c2 = 2.0 * cc * cc - 1.0
        s2 = 2.0 * ss * cc
        c3 = c2 * cc - s2 * ss
        s3 = s2 * cc + c2 * ss
        return c2, s2, c3, s3

    c2w, s2w, c3w, s3w = harmonics(cw, sw)
    c2u, s2u, c3u, s3u = harmonics(cu, su)
    wl = 1.0 - w
    wh = w
    z = jnp.zeros_like(w)
    rows = [wl, wl * cw, wl * sw, wl * c2w, wl * s2w, wl * c3w, wl * s3w, z,
            wh, wh * cu, wh * su, wh * c2u, wh * s2u, wh * c3u, wh * s3u, z]
    wq_ref[...] = jnp.stack(rows, axis=0)


def _make_wq(idx2, *, interpret=False):
    nrow = N // WCOL
    return pl.pallas_call(
        _wq_body,
        grid=(nrow // WR,),
        in_specs=[pl.BlockSpec((WR, WCOL), lambda i: (i, 0))],
        out_specs=[
            pl.BlockSpec((WR, WCOL), lambda i: (i, 0)),
            pl.BlockSpec((16, WR, WCOL), lambda i: (0, i, 0)),
        ],
        out_shape=[
            jax.ShapeDtypeStruct((nrow, WCOL), jnp.int32),
            jax.ShapeDtypeStruct((16, nrow, WCOL), jnp.float32),
        ],
        interpret=interpret,
    )(idx2)


NW = 32
NT = N // NW
CH = 512
NCHUNK = NT // CH
NG = CH // 16
NB = CH // 128


def _sc_body(table, lowidx, wq, out_hbm,
             idxlo_v, rows_v0, rows_v1, wq_v0, wq_v1, out_v0, out_v1,
             gsem0, gsem1, osem0, osem1):
    wid = lax.axis_index("s") * 2 + lax.axis_index("c")
    tbase = wid * NT
    trows = NT // 128
    pltpu.sync_copy(lowidx.at[pl.ds(wid * trows, trows)], idxlo_v)

    slots = ((rows_v0, wq_v0, out_v0, gsem0, osem0),
             (rows_v1, wq_v1, out_v1, gsem1, osem1))

    def in_copies(ci, slot):
        rows_v, wq_v, _, gsem, _ = slots[slot]
        base = tbase + ci * CH
        cps = [pltpu.make_async_copy(wq.at[:, pl.ds(base, CH)], wq_v, gsem)]
        for j in range(NB):
            cps.append(pltpu.make_async_copy(
                table.at[idxlo_v.at[ci * NB + j]],
                rows_v.at[pl.ds(j * 128, 128)], gsem))
        return cps

    def fire_in(ci, slot):
        for cp in in_copies(ci, slot):
            cp.start()

    def wait_in(ci, slot):
        for cp in in_copies(ci, slot):
            cp.wait()

    def out_copy(ci, slot):
        _, _, out_v, _, osem = slots[slot]
        base = tbase + ci * CH
        return pltpu.make_async_copy(
            out_v, out_hbm.at[:, pl.ds(base, CH)], osem)

    def chunk_compute(ci, slot):
        rows_v, wq_v, out_v, _, _ = slots[slot]

        @plsc.parallel_loop(0, NG)
        def group(g):
            g16 = pl.multiple_of(g * 16, 16)
            iota = lax.iota(jnp.int32, 16)
            riv = iota + g16
            wls = [wq_v[m, pl.ds(g16, 16)] for m in range(K7)]
            whs = [wq_v[8 + m, pl.ds(g16, 16)] for m in range(K7)]
            for d in range(C):
                av = (iota + d) & 7
                av8 = av * 8
                acc = None
                for m in range(K7):
                    pw = plsc.load_gather(rows_v, [riv, av8 + m])
                    bf = plsc.bitcast(pw, jnp.bfloat16)
                    vlo, vhi = plsc.unpack(bf,
                                           format=plsc.PackFormat.INTERLEAVED)
                    t = vlo * wls[m] + vhi * whs[m]
                    acc = t if acc is None else acc + t
                plsc.store_scatter(out_v, [av, riv], acc)

    fire_in(0, 0)

    def pair(ci2, carry):
        for b in (0, 1):
            ci = ci2 * 2 + b
            wait_in(ci, b)
            fire_in(jnp.minimum(ci + 1, NCHUNK - 1), 1 - b)

            @pl.when(ci2 >= 1)
            def _drain_out():
                out_copy(ci, b).wait()

            chunk_compute(ci, b)
            out_copy(ci, b).start()
        return carry

    lax.fori_loop(0, NCHUNK // 2, pair, 0)
    wait_in(NCHUNK - 1, 0)
    out_copy(NCHUNK - 2, 0).wait()
    out_copy(NCHUNK - 1, 1).wait()


def _sc_interp(table, lowidx2, wq4):
    mesh = plsc.VectorSubcoreMesh(core_axis_name="c", subcore_axis_name="s")
    f = functools.partial(
        pl.kernel, mesh=mesh,
        compiler_params=pltpu.CompilerParams(needs_layout_passes=False,
                                             use_tc_tiling_on_sc=False),
        out_type=jax.ShapeDtypeStruct((C, N), jnp.float32),
        scratch_types=[
            pltpu.VMEM((NT // 128, 128), jnp.int32),
            pltpu.VMEM((CH, COLS), jnp.int32),
            pltpu.VMEM((CH, COLS), jnp.int32),
            pltpu.VMEM((16, CH), jnp.float32),
            pltpu.VMEM((16, CH), jnp.float32),
            pltpu.VMEM((C, CH), jnp.float32),
            pltpu.VMEM((C, CH), jnp.float32),
            pltpu.SemaphoreType.DMA,
            pltpu.SemaphoreType.DMA,
            pltpu.SemaphoreType.DMA,
            pltpu.SemaphoreType.DMA,
        ],
    )(_sc_body)
    return f(table, lowidx2, wq4)


def kernel(x, W1, b1, W2, b2, indices):
    xt = jnp.transpose(x[0], (1, 2, 0))
    w1t = jnp.transpose(W1, (2, 1, 0))
    o = np.arange(OUTC)
    cols = (o // K7) * 8 + (o % K7)
    w2t = jnp.transpose(W2, (2, 1, 0))
    w2p = jnp.zeros((3, HID, COLS), jnp.float32).at[:, :, cols].set(w2t)
    b2p = jnp.zeros((COLS,), jnp.float32).at[cols].set(b2)

    table = _make_table(xt, w1t, b1[None, :], w2p, b2p[None, :])
    tablep = _make_pair_table(table)
    low, wq = _make_wq(indices.reshape(N // WCOL, WCOL))
    out = _sc_interp(tablep, low.reshape(N // 128, 128), wq.reshape(16, N))
    return out.reshape(1, C, N // VIEWS, VIEWS)


# speedup vs baseline: 1.2427x; 1.0683x over previous
import functools

import jax
import jax.numpy as jnp
import numpy as np
from jax import lax
from jax.experimental import pallas as pl
from jax.experimental.pallas import tpu as pltpu
from jax.experimental.pallas import tpu_sc as plsc

VIEWS = 128
NDET = 512
C = 8
K7 = 7
VU = VIEWS * NDET
N = 64 * 64 * VIEWS
HID = K7 * C * 2
OUTC = K7 * C
COLS = 64


VB = 8
ROWS = VB * NDET


def _gelu_exact(x):
    return 0.5 * x * (1.0 + lax.erf(x * np.float32(1.0 / np.sqrt(2.0))))


def _conv_body(xt_ref, w1_ref, b1_ref, w2_ref, b2_ref, out_ref):
    x2 = xt_ref[...].reshape(ROWS, C)
    i = lax.broadcasted_iota(jnp.int32, (ROWS, 1), 0)
    first = (i % NDET) == 0
    last = (i % NDET) == (NDET - 1)

    def shifts(v):
        z = jnp.zeros((1, v.shape[1]), jnp.float32)
        vm = jnp.where(first, 0.0, jnp.concatenate([z, v[:-1]], axis=0))
        vp = jnp.where(last, 0.0, jnp.concatenate([v[1:], z], axis=0))
        return vm, vp

    xm, xp = shifts(x2)
    f32 = jnp.float32
    bf = jnp.bfloat16
    xcat = jnp.concatenate([xm, x2, xp], axis=1)
    h = jnp.dot(xcat.astype(bf), w1_ref[...].reshape(3 * C, HID).astype(bf),
                preferred_element_type=f32) + b1_ref[...]
    h = _gelu_exact(h)
    hm, hp = shifts(h)
    hcat = jnp.concatenate([hm, h, hp], axis=1)
    y = jnp.dot(hcat.astype(bf),
                w2_ref[...].reshape(3 * HID, COLS).astype(bf),
                preferred_element_type=f32) + b2_ref[...]
    out_ref[...] = y


def _pack_body(lo_ref, hi_ref, out_ref):
    lob = lax.bitcast_convert_type(lo_ref[...], jnp.uint32)
    hib = lax.bitcast_convert_type(hi_ref[...], jnp.uint32)

    def rb(b):
        return (b + jnp.uint32(0x7FFF) + ((b >> 16) & jnp.uint32(1))) >> 16

    word = (rb(hib) << 16) | rb(lob)
    out_ref[...] = lax.bitcast_convert_type(word, jnp.int32)


def _make_pair_table(table, *, interpret=False):
    tnext = jnp.concatenate([table[1:], table[-1:]], axis=0)
    return pl.pallas_call(
        _pack_body,
        grid=(VIEWS // VB,),
        in_specs=[
            pl.BlockSpec((ROWS, COLS), lambda i: (i, 0)),
            pl.BlockSpec((ROWS, COLS), lambda i: (i, 0)),
        ],
        out_specs=pl.BlockSpec((ROWS, COLS), lambda i: (i, 0)),
        out_shape=jax.ShapeDtypeStruct((VU, COLS), jnp.int32),
        interpret=interpret,
    )(table, tnext)


def _make_table(xt, w1t, b1, w2p, b2p, *, interpret=False):
    return pl.pallas_call(
        _conv_body,
        grid=(VIEWS // VB,),
        in_specs=[
            pl.BlockSpec((VB, NDET, C), lambda i: (i, 0, 0)),
            pl.BlockSpec((3, C, HID), lambda i: (0, 0, 0)),
            pl.BlockSpec((1, HID), lambda i: (0, 0)),
            pl.BlockSpec((3, HID, COLS), lambda i: (0, 0, 0)),
            pl.BlockSpec((1, COLS), lambda i: (0, 0)),
        ],
        out_specs=pl.BlockSpec((ROWS, COLS), lambda i: (i, 0)),
        out_shape=jax.ShapeDtypeStruct((VU, COLS), jnp.float32),
        interpret=interpret,
    )(xt, w1t, b1, w2p, b2p)


WR = 8
WCOL = 4096


def _wq_body(idx_ref, low_ref, wq_ref):
    idx = idx_ref[...]
    f = jnp.floor(idx)
    w = idx - f
    low_ref[...] = f.astype(jnp.int32)
    u = w - 1.0
    cw, sw = jnp.cos(w), jnp.sin(w)
    cu, su = jnp.cos(u), jnp.sin(u)

    def harmonics(cc, ss):
        c2 = 2.0 * cc * cc - 1.0
        s2 = 2.0 * ss * cc
        c3 = c2 * cc - s2 * ss
        s3 = s2 * cc + c2 * ss
        return c2, s2, c3, s3

    c2w, s2w, c3w, s3w = harmonics(cw, sw)
    c2u, s2u, c3u, s3u = harmonics(cu, su)
    wl = 1.0 - w
    wh = w
    lo_rows = [wl, wl * cw, wl * sw, wl * c2w, wl * s2w, wl * c3w, wl * s3w]
    hi_rows = [wh, wh * cu, wh * su, wh * c2u, wh * s2u, wh * c3u, wh * s3u]

    def rb(b):
        return (b + jnp.uint32(0x7FFF) + ((b >> 16) & jnp.uint32(1))) >> 16

    words = [
        lax.bitcast_convert_type(
            (rb(lax.bitcast_convert_type(h_, jnp.uint32)) << 16)
            | rb(lax.bitcast_convert_type(l_, jnp.uint32)), jnp.int32)
        for l_, h_ in zip(lo_rows, hi_rows)]
    wq_ref[...] = jnp.stack(words, axis=0)


def _make_wq(idx2, *, interpret=False):
    nrow = N // WCOL
    return pl.pallas_call(
        _wq_body,
        grid=(nrow // WR,),
        in_specs=[pl.BlockSpec((WR, WCOL), lambda i: (i, 0))],
        out_specs=[
            pl.BlockSpec((WR, WCOL), lambda i: (i, 0)),
            pl.BlockSpec((K7, WR, WCOL), lambda i: (0, i, 0)),
        ],
        out_shape=[
            jax.ShapeDtypeStruct((nrow, WCOL), jnp.int32),
            jax.ShapeDtypeStruct((K7, nrow, WCOL), jnp.int32),
        ],
        interpret=interpret,
    )(idx2)


NW = 32
NT = N // NW
CH = 512
NCHUNK = NT // CH
NG = CH // 16
NB = CH // 128


def _sc_body(table, lowidx, wq, out_hbm,
             idxlo_v, rows_v0, rows_v1, wq_v0, wq_v1, out_v0, out_v1,
             gsem0, gsem1, osem0, osem1):
    wid = lax.axis_index("s") * 2 + lax.axis_index("c")
    tbase = wid * NT
    trows = NT // 128
    pltpu.sync_copy(lowidx.at[pl.ds(wid * trows, trows)], idxlo_v)

    slots = ((rows_v0, wq_v0, out_v0, gsem0, osem0),
             (rows_v1, wq_v1, out_v1, gsem1, osem1))

    def in_copies(ci, slot):
        rows_v, wq_v, _, gsem, _ = slots[slot]
        base = tbase + ci * CH
        cps = [pltpu.make_async_copy(wq.at[:, pl.ds(base, CH)], wq_v, gsem)]
        for j in range(NB):
            cps.append(pltpu.make_async_copy(
                table.at[idxlo_v.at[ci * NB + j]],
                rows_v.at[pl.ds(j * 128, 128)], gsem))
        return cps

    def fire_in(ci, slot):
        for cp in in_copies(ci, slot):
            cp.start()

    def wait_in(ci, slot):
        for cp in in_copies(ci, slot):
            cp.wait()

    def out_copy(ci, slot):
        _, _, out_v, _, osem = slots[slot]
        base = tbase + ci * CH
        return pltpu.make_async_copy(
            out_v, out_hbm.at[:, pl.ds(base, CH)], osem)

    def chunk_compute(ci, slot):
        rows_v, wq_v, out_v, _, _ = slots[slot]

        @plsc.parallel_loop(0, NG)
        def group(g):
            g16 = pl.multiple_of(g * 16, 16)
            iota = lax.iota(jnp.int32, 16)
            riv = iota + g16
            wpairs = [plsc.unpack(
                plsc.bitcast(wq_v[m, pl.ds(g16, 16)], jnp.bfloat16),
                format=plsc.PackFormat.INTERLEAVED) for m in range(K7)]
            wls = [p[0] for p in wpairs]
            whs = [p[1] for p in wpairs]
            for d in range(C):
                av = (iota + d) & 7
                av8 = av * 8
                acc = None
                for m in range(K7):
                    pw = plsc.load_gather(rows_v, [riv, av8 + m])
                    bf = plsc.bitcast(pw, jnp.bfloat16)
                    vlo, vhi = plsc.unpack(bf,
                                           format=plsc.PackFormat.INTERLEAVED)
                    t = vlo * wls[m] + vhi * whs[m]
                    acc = t if acc is None else acc + t
                plsc.store_scatter(out_v, [av, riv], acc)

    fire_in(0, 0)

    def pair(ci2, carry):
        for b in (0, 1):
            ci = ci2 * 2 + b
            wait_in(ci, b)
            fire_in(jnp.minimum(ci + 1, NCHUNK - 1), 1 - b)

            @pl.when(ci2 >= 1)
            def _drain_out():
                out_copy(ci, b).wait()

            chunk_compute(ci, b)
            out_copy(ci, b).start()
        return carry

    lax.fori_loop(0, NCHUNK // 2, pair, 0)
    wait_in(NCHUNK - 1, 0)
    out_copy(NCHUNK - 2, 0).wait()
    out_copy(NCHUNK - 1, 1).wait()


def _sc_interp(table, lowidx2, wq4):
    mesh = plsc.VectorSubcoreMesh(core_axis_name="c", subcore_axis_name="s")
    f = functools.partial(
        pl.kernel, mesh=mesh,
        compiler_params=pltpu.CompilerParams(needs_layout_passes=False,
                                             use_tc_tiling_on_sc=False),
        out_type=jax.ShapeDtypeStruct((C, N), jnp.float32),
        scratch_types=[
            pltpu.VMEM((NT // 128, 128), jnp.int32),
            pltpu.VMEM((CH, COLS), jnp.int32),
            pltpu.VMEM((CH, COLS), jnp.int32),
            pltpu.VMEM((K7, CH), jnp.int32),
            pltpu.VMEM((K7, CH), jnp.int32),
            pltpu.VMEM((C, CH), jnp.float32),
            pltpu.VMEM((C, CH), jnp.float32),
            pltpu.SemaphoreType.DMA,
            pltpu.SemaphoreType.DMA,
            pltpu.SemaphoreType.DMA,
            pltpu.SemaphoreType.DMA,
        ],
    )(_sc_body)
    return f(table, lowidx2, wq4)


def kernel(x, W1, b1, W2, b2, indices):
    xt = jnp.transpose(x[0], (1, 2, 0))
    w1t = jnp.transpose(W1, (2, 1, 0))
    o = np.arange(OUTC)
    cols = (o // K7) * 8 + (o % K7)
    w2t = jnp.transpose(W2, (2, 1, 0))
    w2p = jnp.zeros((3, HID, COLS), jnp.float32).at[:, :, cols].set(w2t)
    b2p = jnp.zeros((COLS,), jnp.float32).at[cols].set(b2)

    table = _make_table(xt, w1t, b1[None, :], w2p, b2p[None, :])
    tablep = _make_pair_table(table)
    low, wq = _make_wq(indices.reshape(N // WCOL, WCOL))
    out = _sc_interp(tablep, low.reshape(N // 128, 128), wq.reshape(K7, N))
    return out.reshape(1, C, N // VIEWS, VIEWS)


# speedup vs baseline: 1.3184x; 1.0610x over previous
import functools

import jax
import jax.numpy as jnp
import numpy as np
from jax import lax
from jax.experimental import pallas as pl
from jax.experimental.pallas import tpu as pltpu
from jax.experimental.pallas import tpu_sc as plsc

VIEWS = 128
NDET = 512
C = 8
K7 = 7
VU = VIEWS * NDET
N = 64 * 64 * VIEWS
HID = K7 * C * 2
OUTC = K7 * C
COLS = 64


VB = 8
ROWS = VB * NDET


def _gelu_exact(x):
    return 0.5 * x * (1.0 + lax.erf(x * np.float32(1.0 / np.sqrt(2.0))))


def _conv_body(xt_ref, w1_ref, b1_ref, w2_ref, b2_ref, out_ref):
    x2 = xt_ref[...].reshape(ROWS, C)
    i = lax.broadcasted_iota(jnp.int32, (ROWS, 1), 0)
    first = (i % NDET) == 0
    last = (i % NDET) == (NDET - 1)

    def shifts(v):
        z = jnp.zeros((1, v.shape[1]), jnp.float32)
        vm = jnp.where(first, 0.0, jnp.concatenate([z, v[:-1]], axis=0))
        vp = jnp.where(last, 0.0, jnp.concatenate([v[1:], z], axis=0))
        return vm, vp

    xm, xp = shifts(x2)
    f32 = jnp.float32
    bf = jnp.bfloat16
    xcat = jnp.concatenate([xm, x2, xp], axis=1)
    h = jnp.dot(xcat.astype(bf), w1_ref[...].reshape(3 * C, HID).astype(bf),
                preferred_element_type=f32) + b1_ref[...]
    h = _gelu_exact(h)
    hm, hp = shifts(h)
    hcat = jnp.concatenate([hm, h, hp], axis=1)
    y = jnp.dot(hcat.astype(bf),
                w2_ref[...].reshape(3 * HID, COLS).astype(bf),
                preferred_element_type=f32) + b2_ref[...]
    out_ref[...] = y


def _pack_body(lo_ref, nxt_ref, out_ref):
    lo = lo_ref[...]
    hi = jnp.concatenate([lo[1:], nxt_ref[...].reshape(1, COLS)], axis=0)
    lob = lax.bitcast_convert_type(lo, jnp.uint32)
    hib = lax.bitcast_convert_type(hi, jnp.uint32)

    def rb(b):
        return (b + jnp.uint32(0x7FFF) + ((b >> 16) & jnp.uint32(1))) >> 16

    word = (rb(hib) << 16) | rb(lob)
    out_ref[...] = lax.bitcast_convert_type(word, jnp.int32)


def _make_pair_table(table, *, interpret=False):
    nxt = jnp.concatenate([table[ROWS::ROWS], table[-1:]], axis=0)
    nxt = nxt.reshape(VIEWS // VB, 1, COLS)
    return pl.pallas_call(
        _pack_body,
        grid=(VIEWS // VB,),
        in_specs=[
            pl.BlockSpec((ROWS, COLS), lambda i: (i, 0)),
            pl.BlockSpec((1, 1, COLS), lambda i: (i, 0, 0)),
        ],
        out_specs=pl.BlockSpec((ROWS, COLS), lambda i: (i, 0)),
        out_shape=jax.ShapeDtypeStruct((VU, COLS), jnp.int32),
        interpret=interpret,
    )(table, nxt)


def _make_table(xt, w1t, b1, w2p, b2p, *, interpret=False):
    return pl.pallas_call(
        _conv_body,
        grid=(VIEWS // VB,),
        in_specs=[
            pl.BlockSpec((VB, NDET, C), lambda i: (i, 0, 0)),
            pl.BlockSpec((3, C, HID), lambda i: (0, 0, 0)),
            pl.BlockSpec((1, HID), lambda i: (0, 0)),
            pl.BlockSpec((3, HID, COLS), lambda i: (0, 0, 0)),
            pl.BlockSpec((1, COLS), lambda i: (0, 0)),
        ],
        out_specs=pl.BlockSpec((ROWS, COLS), lambda i: (i, 0)),
        out_shape=jax.ShapeDtypeStruct((VU, COLS), jnp.float32),
        interpret=interpret,
    )(xt, w1t, b1, w2p, b2p)


WR = 8
WCOL = 4096


def _wq_body(idx_ref, low_ref, wq_ref):
    idx = idx_ref[...]
    f = jnp.floor(idx)
    w = idx - f
    low_ref[...] = f.astype(jnp.int32)
    u = w - 1.0
    cw, sw = jnp.cos(w), jnp.sin(w)
    cu, su = jnp.cos(u), jnp.sin(u)

    def harmonics(cc, ss):
        c2 = 2.0 * cc * cc - 1.0
        s2 = 2.0 * ss * cc
        c3 = c2 * cc - s2 * ss
        s3 = s2 * cc + c2 * ss
        return c2, s2, c3, s3

    c2w, s2w, c3w, s3w = harmonics(cw, sw)
    c2u, s2u, c3u, s3u = harmonics(cu, su)
    wl = 1.0 - w
    wh = w
    lo_rows = [wl, wl * cw, wl * sw, wl * c2w, wl * s2w, wl * c3w, wl * s3w]
    hi_rows = [wh, wh * cu, wh * su, wh * c2u, wh * s2u, wh * c3u, wh * s3u]

    def rb(b):
        return (b + jnp.uint32(0x7FFF) + ((b >> 16) & jnp.uint32(1))) >> 16

    words = [
        lax.bitcast_convert_type(
            (rb(lax.bitcast_convert_type(h_, jnp.uint32)) << 16)
            | rb(lax.bitcast_convert_type(l_, jnp.uint32)), jnp.int32)
        for l_, h_ in zip(lo_rows, hi_rows)]
    wq_ref[...] = jnp.stack(words, axis=0)


def _make_wq(idx2, *, interpret=False):
    nrow = N // WCOL
    return pl.pallas_call(
        _wq_body,
        grid=(nrow // WR,),
        in_specs=[pl.BlockSpec((WR, WCOL), lambda i: (i, 0))],
        out_specs=[
            pl.BlockSpec((WR, WCOL), lambda i: (i, 0)),
            pl.BlockSpec((K7, WR, WCOL), lambda i: (0, i, 0)),
        ],
        out_shape=[
            jax.ShapeDtypeStruct((nrow, WCOL), jnp.int32),
            jax.ShapeDtypeStruct((K7, nrow, WCOL), jnp.int32),
        ],
        interpret=interpret,
    )(idx2)


NW = 32
NT = N // NW
CH = 512
NCHUNK = NT // CH
NG = CH // 16
NB = CH // 128


def _sc_body(table, lowidx, wq, out_hbm,
             idxlo_v, rows_v0, rows_v1, wq_v0, wq_v1, out_v0, out_v1,
             gsem0, gsem1, osem0, osem1):
    wid = lax.axis_index("s") * 2 + lax.axis_index("c")
    tbase = wid * NT
    trows = NT // 128
    pltpu.sync_copy(lowidx.at[pl.ds(wid * trows, trows)], idxlo_v)

    slots = ((rows_v0, wq_v0, out_v0, gsem0, osem0),
             (rows_v1, wq_v1, out_v1, gsem1, osem1))

    def in_copies(ci, slot):
        rows_v, wq_v, _, gsem, _ = slots[slot]
        base = tbase + ci * CH
        cps = [pltpu.make_async_copy(wq.at[:, pl.ds(base, CH)], wq_v, gsem)]
        for j in range(NB):
            cps.append(pltpu.make_async_copy(
                table.at[idxlo_v.at[ci * NB + j]],
                rows_v.at[pl.ds(j * 128, 128)], gsem))
        return cps

    def fire_in(ci, slot):
        for cp in in_copies(ci, slot):
            cp.start()

    def wait_in(ci, slot):
        for cp in in_copies(ci, slot):
            cp.wait()

    def out_copy(ci, slot):
        _, _, out_v, _, osem = slots[slot]
        base = tbase + ci * CH
        return pltpu.make_async_copy(
            out_v, out_hbm.at[:, pl.ds(base, CH)], osem)

    def chunk_compute(ci, slot):
        rows_v, wq_v, out_v, _, _ = slots[slot]

        @plsc.parallel_loop(0, NG)
        def group(g):
            g16 = pl.multiple_of(g * 16, 16)
            iota = lax.iota(jnp.int32, 16)
            riv = iota + g16
            wpairs = [plsc.unpack(
                plsc.bitcast(wq_v[m, pl.ds(g16, 16)], jnp.bfloat16),
                format=plsc.PackFormat.INTERLEAVED) for m in range(K7)]
            wls = [p[0] for p in wpairs]
            whs = [p[1] for p in wpairs]
            for d in range(C):
                av = (iota + d) & 7
                av8 = av * 8
                acc = None
                for m in range(K7):
                    pw = plsc.load_gather(rows_v, [riv, av8 + m])
                    bf = plsc.bitcast(pw, jnp.bfloat16)
                    vlo, vhi = plsc.unpack(bf,
                                           format=plsc.PackFormat.INTERLEAVED)
                    t = vlo * wls[m] + vhi * whs[m]
                    acc = t if acc is None else acc + t
                plsc.store_scatter(out_v, [av, riv], acc)

    fire_in(0, 0)

    def pair(ci2, carry):
        for b in (0, 1):
            ci = ci2 * 2 + b
            wait_in(ci, b)
            fire_in(jnp.minimum(ci + 1, NCHUNK - 1), 1 - b)

            @pl.when(ci2 >= 1)
            def _drain_out():
                out_copy(ci, b).wait()

            chunk_compute(ci, b)
            out_copy(ci, b).start()
        return carry

    lax.fori_loop(0, NCHUNK // 2, pair, 0)
    wait_in(NCHUNK - 1, 0)
    out_copy(NCHUNK - 2, 0).wait()
    out_copy(NCHUNK - 1, 1).wait()


def _sc_interp(table, lowidx2, wq4):
    mesh = plsc.VectorSubcoreMesh(core_axis_name="c", subcore_axis_name="s")
    f = functools.partial(
        pl.kernel, mesh=mesh,
        compiler_params=pltpu.CompilerParams(needs_layout_passes=False,
                                             use_tc_tiling_on_sc=False),
        out_type=jax.ShapeDtypeStruct((C, N), jnp.float32),
        scratch_types=[
            pltpu.VMEM((NT // 128, 128), jnp.int32),
            pltpu.VMEM((CH, COLS), jnp.int32),
            pltpu.VMEM((CH, COLS), jnp.int32),
            pltpu.VMEM((K7, CH), jnp.int32),
            pltpu.VMEM((K7, CH), jnp.int32),
            pltpu.VMEM((C, CH), jnp.float32),
            pltpu.VMEM((C, CH), jnp.float32),
            pltpu.SemaphoreType.DMA,
            pltpu.SemaphoreType.DMA,
            pltpu.SemaphoreType.DMA,
            pltpu.SemaphoreType.DMA,
        ],
    )(_sc_body)
    return f(table, lowidx2, wq4)


def kernel(x, W1, b1, W2, b2, indices):
    xt = jnp.transpose(x[0], (1, 2, 0))
    w1t = jnp.transpose(W1, (2, 1, 0))
    o = np.arange(OUTC)
    cols = (o // K7) * 8 + (o % K7)
    w2t = jnp.transpose(W2, (2, 1, 0))
    w2p = jnp.zeros((3, HID, COLS), jnp.float32).at[:, :, cols].set(w2t)
    b2p = jnp.zeros((COLS,), jnp.float32).at[cols].set(b2)

    table = _make_table(xt, w1t, b1[None, :], w2p, b2p[None, :])
    tablep = _make_pair_table(table)
    low, wq = _make_wq(indices.reshape(N // WCOL, WCOL))
    out = _sc_interp(tablep, low.reshape(N // 128, 128), wq.reshape(K7, N))
    return out.reshape(1, C, N // VIEWS, VIEWS)
